# Initial kernel scaffold; baseline (speedup 1.0000x reference)
import jax
import jax.numpy as jnp
from jax.experimental import pallas as pl


def kernel(feat, edge_index, theta_W, theta_b, phi_W, phi_b):
    raise NotImplementedError("write your pallas kernel here")


# speedup vs baseline: 1.7422x; 1.7422x over previous
import functools

import jax
import jax.numpy as jnp
from jax import lax
from jax.experimental import pallas as pl
from jax.experimental.pallas import tpu as pltpu
from jax.experimental.pallas import tpu_sc as plsc

N_NODES = 10000
N_EDGES = 320000
D = 256

NC = 2
NS = 16
NW = NC * NS
ROWS_PER = 320
N_PAD = NW * ROWS_PER
DUMMY = ROWS_PER
ACC_ROWS = ROWS_PER + 8
CHUNK = 2000
N_CHUNKS = N_EDGES // CHUNK
VREGS = CHUNK // 16
CAP = 4352
FLUSH_AT = 2208
G = 64


def _linear_block(x_ref, w_ref, tb_ref, pb_ref, o_ref):
    acc = lax.dot_general(x_ref[...], w_ref[...],
                          (((1,), (1,)), ((), ())),
                          preferred_element_type=jnp.float32)
    o_ref[...] = acc + tb_ref[...] + pb_ref[...]


def _linear(feat, phi_W, theta_b2, phi_b2):
    M = feat.shape[0]
    BM = 2000
    return pl.pallas_call(
        _linear_block,
        grid=(M // BM,),
        in_specs=[
            pl.BlockSpec((BM, D), lambda i: (i, 0)),
            pl.BlockSpec((D, D), lambda i: (0, 0)),
            pl.BlockSpec((1, D), lambda i: (0, 0)),
            pl.BlockSpec((1, D), lambda i: (0, 0)),
        ],
        out_specs=pl.BlockSpec((BM, D), lambda i: (i, 0)),
        out_shape=jax.ShapeDtypeStruct((M, D), jnp.float32),
    )(feat, phi_W, theta_b2, phi_b2)


def _sc_segment_max(h, src, dst):
    mesh = plsc.VectorSubcoreMesh(core_axis_name="c", subcore_axis_name="s")

    @functools.partial(
        pl.kernel,
        mesh=mesh,
        out_type=jax.ShapeDtypeStruct((N_PAD, D), jnp.float32),
        compiler_params=pltpu.CompilerParams(needs_layout_passes=False),
        scratch_types=[
            pltpu.VMEM((ACC_ROWS, D), jnp.float32),
            pltpu.VMEM((CHUNK,), jnp.int32),
            pltpu.VMEM((CHUNK,), jnp.int32),
            pltpu.VMEM((CAP,), jnp.int32),
            pltpu.VMEM((CAP,), jnp.int32),
            pltpu.VMEM((G, D), jnp.float32),
            pltpu.SemaphoreType.DMA,
        ],
    )
    def seg_max(h_hbm, src_hbm, dst_hbm, out_hbm,
                acc, srcb, dstb, cidx, cdst, rows, sem):
        wid = lax.axis_index("s") * NC + lax.axis_index("c")
        base = (wid * ROWS_PER).astype(jnp.int32)

        neg = jnp.full((16,), -jnp.inf, jnp.float32)
        zero = jnp.zeros((16,), jnp.float32)

        def init_row(r, _):
            for c in range(D // 16):
                acc[r, pl.ds(c * 16, 16)] = neg
            return 0
        lax.fori_loop(0, ACC_ROWS, init_row, 0)

        pad_src = jnp.broadcast_to(wid.astype(jnp.int32), (16,))
        pad_dst = jnp.full((16,), DUMMY, jnp.int32)

        def flush(cnt):
            for kk in range(G // 16):
                cidx[pl.ds(cnt + kk * 16, 16)] = pad_src
                cdst[pl.ds(cnt + kk * 16, 16)] = pad_dst
            n_sub = (cnt + (G - 1)) // G

            def sub(j, _):
                off = j * G
                pltpu.async_copy(h_hbm.at[cidx.at[pl.ds(off, G)]],
                                 rows, sem).wait()

                def edge(e, _):
                    drow = cdst[pl.ds(off + e, 16)][0]
                    for c in range(D // 16):
                        sl = pl.ds(c * 16, 16)
                        acc[drow, sl] = jnp.maximum(acc[drow, sl],
                                                    rows[e, sl])
                    return 0
                lax.fori_loop(0, G, edge, 0)
                return 0
            lax.fori_loop(0, n_sub, sub, 0)

        def chunk_body(i, cnt):
            eoff = i * CHUNK
            pltpu.sync_copy(src_hbm.at[pl.ds(eoff, CHUNK)], srcb)
            pltpu.sync_copy(dst_hbm.at[pl.ds(eoff, CHUNK)], dstb)

            def vec(v, c):
                sl = pl.ds(v * 16, 16)
                s = srcb[sl]
                d = dstb[sl]
                m = (d >= base) & (d < base + ROWS_PER)
                plsc.store_compressed(cidx.at[pl.ds(c, 16)], s, mask=m)
                plsc.store_compressed(cdst.at[pl.ds(c, 16)], d - base, mask=m)
                return c + jnp.sum(m.astype(jnp.int32))
            cnt = lax.fori_loop(0, VREGS, vec, cnt)

            def do_flush(c):
                flush(c)
                return jnp.int32(0)
            return lax.cond(cnt >= FLUSH_AT, do_flush, lambda c: c, cnt)

        cnt = lax.fori_loop(0, N_CHUNKS, chunk_body, jnp.int32(0))

        def final_flush(c):
            flush(c)
            return jnp.int32(0)
        lax.cond(cnt > 0, final_flush, lambda c: c, cnt)

        def fin_row(r, _):
            for c in range(D // 16):
                sl = pl.ds(c * 16, 16)
                v = acc[r, sl]
                acc[r, sl] = jnp.where(v == neg, zero, v)
            return 0
        lax.fori_loop(0, ROWS_PER, fin_row, 0)

        pltpu.sync_copy(acc.at[pl.ds(0, ROWS_PER)],
                        out_hbm.at[pl.ds(base, ROWS_PER)])

    return seg_max(h, src, dst)


def kernel(feat, edge_index, theta_W, theta_b, phi_W, phi_b):
    theta_b2 = theta_b.reshape(1, D).astype(jnp.float32)
    phi_b2 = phi_b.reshape(1, D).astype(jnp.float32)
    h = _linear(feat, phi_W, theta_b2, phi_b2)
    ei = edge_index.astype(jnp.int32)
    out_pad = _sc_segment_max(h, ei[0], ei[1])
    return out_pad[:N_NODES]


# speedup vs baseline: 2.0532x; 1.1786x over previous
import functools

import jax
import jax.numpy as jnp
from jax import lax
from jax.experimental import pallas as pl
from jax.experimental.pallas import tpu as pltpu
from jax.experimental.pallas import tpu_sc as plsc

N_NODES = 10000
N_EDGES = 320000
D = 256

NC = 2
NS = 16
NW = NC * NS
ROWS_PER = 320
N_PAD = NW * ROWS_PER
DUMMY = ROWS_PER
ACC_ROWS = ROWS_PER + 8
CHUNK = 2000
N_CHUNKS = N_EDGES // CHUNK
VREGS = CHUNK // 16
CAP = 4352
FLUSH_AT = 2208
G = 56
SCAN_UNROLL = 5


def _linear_block(x_ref, w_ref, tb_ref, pb_ref, o_ref):
    acc = lax.dot_general(x_ref[...], w_ref[...],
                          (((1,), (1,)), ((), ())),
                          preferred_element_type=jnp.float32)
    o_ref[...] = acc + tb_ref[...] + pb_ref[...]


def _linear(feat, phi_W, theta_b2, phi_b2):
    M = feat.shape[0]
    BM = 2000
    return pl.pallas_call(
        _linear_block,
        grid=(M // BM,),
        in_specs=[
            pl.BlockSpec((BM, D), lambda i: (i, 0)),
            pl.BlockSpec((D, D), lambda i: (0, 0)),
            pl.BlockSpec((1, D), lambda i: (0, 0)),
            pl.BlockSpec((1, D), lambda i: (0, 0)),
        ],
        out_specs=pl.BlockSpec((BM, D), lambda i: (i, 0)),
        out_shape=jax.ShapeDtypeStruct((M, D), jnp.float32),
    )(feat, phi_W, theta_b2, phi_b2)


def _sc_segment_max(h, src, dst):
    mesh = plsc.VectorSubcoreMesh(core_axis_name="c", subcore_axis_name="s")

    @functools.partial(
        pl.kernel,
        mesh=mesh,
        out_type=jax.ShapeDtypeStruct((N_PAD, D), jnp.float32),
        compiler_params=pltpu.CompilerParams(needs_layout_passes=False),
        scratch_types=[
            pltpu.VMEM((ACC_ROWS, D), jnp.float32),
            pltpu.VMEM((CHUNK,), jnp.int32),
            pltpu.VMEM((CHUNK,), jnp.int32),
            pltpu.VMEM((CAP,), jnp.int32),
            pltpu.VMEM((CAP,), jnp.int32),
            pltpu.VMEM((2, G, D), jnp.float32),
            pltpu.SemaphoreType.DMA,
        ],
    )
    def seg_max(h_hbm, src_hbm, dst_hbm, out_hbm,
                acc, srcb, dstb, cidx, cdst, rows, sem):
        wid = lax.axis_index("s") * NC + lax.axis_index("c")
        base = (wid * ROWS_PER).astype(jnp.int32)

        neg = jnp.full((16,), -jnp.inf, jnp.float32)
        zero = jnp.zeros((16,), jnp.float32)

        def init_row(r, _):
            for c in range(D // 16):
                acc[r, pl.ds(c * 16, 16)] = neg
            return 0
        lax.fori_loop(0, ACC_ROWS, init_row, 0)

        pad_src = jnp.broadcast_to(wid.astype(jnp.int32), (16,))
        pad_dst = jnp.full((16,), DUMMY, jnp.int32)

        def flush(cnt):
            for kk in range(-(-G // 16) + 1):
                cidx[pl.ds(cnt + kk * 16, 16)] = pad_src
                cdst[pl.ds(cnt + kk * 16, 16)] = pad_dst
            n_sub = (cnt + (G - 1)) // G

            pltpu.async_copy(h_hbm.at[cidx.at[pl.ds(0, G)]], rows.at[0], sem)

            def sub(j, _):
                jb = j & 1
                pltpu.make_async_copy(h_hbm.at[cidx.at[pl.ds(0, G)]],
                                      rows.at[jb], sem).wait()

                @pl.when(j + 1 < n_sub)
                def _():
                    pltpu.async_copy(
                        h_hbm.at[cidx.at[pl.ds((j + 1) * G, G)]],
                        rows.at[1 - jb], sem)

                off = j * G

                def edge(e, _):
                    drow = cdst[pl.ds(off + e, 16)][0]
                    for c in range(D // 16):
                        sl = pl.ds(c * 16, 16)
                        acc[drow, sl] = jnp.maximum(acc[drow, sl],
                                                    rows[jb, e, sl])
                    return 0
                lax.fori_loop(0, G, edge, 0)
                return 0
            lax.fori_loop(0, n_sub, sub, 0)

        def chunk_body(i, cnt):
            eoff = i * CHUNK
            pltpu.sync_copy(src_hbm.at[pl.ds(eoff, CHUNK)], srcb)
            pltpu.sync_copy(dst_hbm.at[pl.ds(eoff, CHUNK)], dstb)

            def vec(v, c):
                for u in range(SCAN_UNROLL):
                    sl = pl.ds((v * SCAN_UNROLL + u) * 16, 16)
                    s = srcb[sl]
                    d = dstb[sl]
                    m = (d >= base) & (d < base + ROWS_PER)
                    plsc.store_compressed(cidx.at[pl.ds(c, 16)], s, mask=m)
                    plsc.store_compressed(cdst.at[pl.ds(c, 16)], d - base,
                                          mask=m)
                    c = c + plsc.all_reduce_population_count(m)[0]
                return c
            cnt = lax.fori_loop(0, VREGS // SCAN_UNROLL, vec, cnt)

            def do_flush(c):
                flush(c)
                return jnp.int32(0)
            return lax.cond(cnt >= FLUSH_AT, do_flush, lambda c: c, cnt)

        cnt = lax.fori_loop(0, N_CHUNKS, chunk_body, jnp.int32(0))

        def final_flush(c):
            flush(c)
            return jnp.int32(0)
        lax.cond(cnt > 0, final_flush, lambda c: c, cnt)

        def fin_row(r, _):
            for c in range(D // 16):
                sl = pl.ds(c * 16, 16)
                v = acc[r, sl]
                acc[r, sl] = jnp.where(v == neg, zero, v)
            return 0
        lax.fori_loop(0, ROWS_PER, fin_row, 0)

        pltpu.sync_copy(acc.at[pl.ds(0, ROWS_PER)],
                        out_hbm.at[pl.ds(base, ROWS_PER)])

    return seg_max(h, src, dst)


def kernel(feat, edge_index, theta_W, theta_b, phi_W, phi_b):
    theta_b2 = theta_b.reshape(1, D).astype(jnp.float32)
    phi_b2 = phi_b.reshape(1, D).astype(jnp.float32)
    h = _linear(feat, phi_W, theta_b2, phi_b2)
    ei = edge_index.astype(jnp.int32)
    out_pad = _sc_segment_max(h, ei[0], ei[1])
    return out_pad[:N_NODES]
